# baseline (speedup 1.0000x reference)
import jax
import jax.numpy as jnp
from jax import lax
from jax.experimental import pallas as pl
from jax.experimental.pallas import tpu as pltpu
from jax.experimental.pallas import tpu_sc as plsc

N = 10000
E2 = 80000
E = 160000
AF = 133
BFD = 147
H = 128
M = 512
T = 2
DEPTH = 3

_INTERP = False


def _bonds_body(x_ref, w_ref, b_ref, inp_ref, msg_ref):
    t = jnp.dot(x_ref[...], w_ref[...], preferred_element_type=jnp.float32)
    t = t + b_ref[...]
    inp_ref[...] = t
    msg_ref[...] = jnp.maximum(t, 0.0)


def _bonds_mm(x, w, b):
    blk = 1280
    r = x.shape[0]
    return pl.pallas_call(
        _bonds_body,
        grid=(r // blk,),
        in_specs=[
            pl.BlockSpec((blk, BFD), lambda i: (i, 0)),
            pl.BlockSpec((BFD, H), lambda i: (0, 0)),
            pl.BlockSpec((1, H), lambda i: (0, 0)),
        ],
        out_specs=[
            pl.BlockSpec((blk, H), lambda i: (i, 0)),
            pl.BlockSpec((blk, H), lambda i: (i, 0)),
        ],
        out_shape=[jax.ShapeDtypeStruct((r, H), jnp.float32)] * 2,
        interpret=_INTERP,
    )(x, w, b)


def _add2_body(p_ref, o_ref):
    o_ref[...] = p_ref[0] + p_ref[1]


def _add2(p):
    rows = p.shape[1]
    blk = 1264
    return pl.pallas_call(
        _add2_body,
        grid=(rows // blk,),
        in_specs=[pl.BlockSpec((2, blk, H), lambda i: (0, i, 0))],
        out_specs=pl.BlockSpec((blk, H), lambda i: (i, 0)),
        out_shape=jax.ShapeDtypeStruct((rows, H), jnp.float32),
        interpret=_INTERP,
    )(p)


def _iter_body(g_ref, m_ref, w_ref, inp_ref, b_ref, o_ref):
    t = g_ref[...] - m_ref[...]
    u = jnp.dot(t, w_ref[...], preferred_element_type=jnp.float32)
    o_ref[...] = jnp.maximum(inp_ref[...] + u + b_ref[...], 0.0)


def _iter_mm(g, msg, w, inp, b):
    blk = 1600
    nblk = E // blk
    half = E2 // blk
    sw = lambda i: ((i + half) % nblk, 0)
    return pl.pallas_call(
        _iter_body,
        grid=(nblk,),
        in_specs=[
            pl.BlockSpec((blk, H), lambda i: (i, 0)),
            pl.BlockSpec((blk, H), lambda i: (i, 0)),
            pl.BlockSpec((H, H), lambda i: (0, 0)),
            pl.BlockSpec((blk, H), sw),
            pl.BlockSpec((1, H), lambda i: (0, 0)),
        ],
        out_specs=pl.BlockSpec((blk, H), sw),
        out_shape=jax.ShapeDtypeStruct((E, H), jnp.float32),
        interpret=_INTERP,
    )(g, msg, w, inp, b)


def _out_body(fa_ref, am_ref, w1_ref, w2_ref, b_ref, o_ref):
    t = jnp.dot(fa_ref[...], w1_ref[...], preferred_element_type=jnp.float32)
    am = am_ref[0] + am_ref[1]
    t = t + jnp.dot(am, w2_ref[...], preferred_element_type=jnp.float32)
    o_ref[...] = jnp.maximum(t + b_ref[...], 0.0)


def _out_mm(fa, am, w1, w2, b):
    blk = 1000
    return pl.pallas_call(
        _out_body,
        grid=(N // blk,),
        in_specs=[
            pl.BlockSpec((blk, AF), lambda i: (i, 0)),
            pl.BlockSpec((2, blk, H), lambda i: (0, i, 0)),
            pl.BlockSpec((AF, H), lambda i: (0, 0)),
            pl.BlockSpec((H, H), lambda i: (0, 0)),
            pl.BlockSpec((1, H), lambda i: (0, 0)),
        ],
        out_specs=pl.BlockSpec((blk, H), lambda i: (i, 0)),
        out_shape=jax.ShapeDtypeStruct((N, H), jnp.float32),
        interpret=_INTERP,
    )(fa, am, w1, w2, b)


def _diff_body(ap_ref, ar_ref, w_ref, b_ref, d_ref, inp_ref, m_ref):
    d = ap_ref[...] - ar_ref[...]
    t = jnp.dot(d, w_ref[...], preferred_element_type=jnp.float32) + b_ref[...]
    d_ref[...] = d
    inp_ref[...] = t
    m_ref[...] = jnp.maximum(t, 0.0)


def _diff_mm(ap, ar, w, b):
    blk = 1000
    return pl.pallas_call(
        _diff_body,
        grid=(N // blk,),
        in_specs=[
            pl.BlockSpec((blk, H), lambda i: (i, 0)),
            pl.BlockSpec((blk, H), lambda i: (i, 0)),
            pl.BlockSpec((H, H), lambda i: (0, 0)),
            pl.BlockSpec((1, H), lambda i: (0, 0)),
        ],
        out_specs=[
            pl.BlockSpec((blk, H), lambda i: (i, 0)),
            pl.BlockSpec((blk, H), lambda i: (i, 0)),
            pl.BlockSpec((blk, H), lambda i: (i, 0)),
        ],
        out_shape=[jax.ShapeDtypeStruct((N, H), jnp.float32)] * 3,
        interpret=_INTERP,
    )(ap, ar, w, b)


def _diter_body(agg_ref, w_ref, inp_ref, b_ref, m_ref):
    a = agg_ref[0] + agg_ref[1]
    u = jnp.dot(a, w_ref[...], preferred_element_type=jnp.float32)
    m_ref[...] = jnp.maximum(inp_ref[...] + u + b_ref[...], 0.0)


def _diter_mm(agg, w, inp, b):
    blk = 1000
    return pl.pallas_call(
        _diter_body,
        grid=(N // blk,),
        in_specs=[
            pl.BlockSpec((2, blk, H), lambda i: (0, i, 0)),
            pl.BlockSpec((H, H), lambda i: (0, 0)),
            pl.BlockSpec((blk, H), lambda i: (i, 0)),
            pl.BlockSpec((1, H), lambda i: (0, 0)),
        ],
        out_specs=pl.BlockSpec((blk, H), lambda i: (i, 0)),
        out_shape=jax.ShapeDtypeStruct((N, H), jnp.float32),
        interpret=_INTERP,
    )(agg, w, inp, b)


def _hid_body(d_ref, m_ref, w1_ref, w2_ref, b_ref, h_ref):
    t = jnp.dot(d_ref[...], w1_ref[...], preferred_element_type=jnp.float32)
    t = t + jnp.dot(m_ref[...], w2_ref[...], preferred_element_type=jnp.float32)
    h_ref[...] = jnp.maximum(t + b_ref[...], 0.0)


def _hid_mm(d, msg, w1, w2, b, rpad):
    blk = 1000
    return pl.pallas_call(
        _hid_body,
        grid=(N // blk,),
        in_specs=[
            pl.BlockSpec((blk, H), lambda i: (i, 0)),
            pl.BlockSpec((blk, H), lambda i: (i, 0)),
            pl.BlockSpec((H, H), lambda i: (0, 0)),
            pl.BlockSpec((H, H), lambda i: (0, 0)),
            pl.BlockSpec((1, H), lambda i: (0, 0)),
        ],
        out_specs=pl.BlockSpec((blk, H), lambda i: (i, 0)),
        out_shape=jax.ShapeDtypeStruct((rpad, H), jnp.float32),
        interpret=_INTERP,
    )(d, msg, w1, w2, b)


def _ffn_body(s1_ref, s2_ref, c_ref, w1_ref, b1_ref, w2_ref, b2_ref,
              w3_ref, b3_ref, o_ref):
    r = 1.0 / jnp.maximum(c_ref[0] + c_ref[1], 1.0)

    def head(s_ref):
        m = (s_ref[0] + s_ref[1]) * r
        h1 = jnp.dot(m, w1_ref[...], preferred_element_type=jnp.float32)
        h1 = jnp.maximum(h1 + b1_ref[...], 0.0)
        h2 = jnp.dot(h1, w2_ref[...], preferred_element_type=jnp.float32)
        h2 = jnp.maximum(h2 + b2_ref[...], 0.0)
        h3 = jnp.dot(h2, w3_ref[...], preferred_element_type=jnp.float32)
        return jnp.clip(h3 + b3_ref[...], 0.0, 6.0)

    o_ref[...] = jax.nn.sigmoid(head(s1_ref) - head(s2_ref))


def _ffn(s1, s2, cnt, w1, b1, w2, b2, w3, b3):
    full = lambda a, b: pl.BlockSpec((a, b), lambda: (0, 0))
    st = pl.BlockSpec((2, M, H), lambda: (0, 0, 0))
    return pl.pallas_call(
        _ffn_body,
        in_specs=[
            st, st, st,
            full(H, H), full(1, H), full(H, H), full(1, H),
            full(H, H), full(1, H),
        ],
        out_specs=full(M, H),
        out_shape=jax.ShapeDtypeStruct((M, H), jnp.float32),
        interpret=_INTERP,
    )(s1, s2, cnt, w1, b1, w2, b2, w3, b3)


_NC = 2
_NS = 16
_CP = 128
_ET = E // (_NC * _NS)
_NF = _ET // _CP
_TAIL = _ET - _NF * _CP
_NCH = _NF + 1
_ZR = 632
_NP = _NS * _ZR
_AT = 384
_NHP = _NC * _NS * _AT
_MT = M // _NS

_MESH = plsc.VectorSubcoreMesh(
    core_axis_name="c", subcore_axis_name="s",
    num_cores=_NC, num_subcores=_NS)


def _pad_edge_idx(idx, padval):
    a = idx.astype(jnp.int32).reshape(_NC * _NS, _ET)
    pad = jnp.full((_NC * _NS, _NCH * _CP - _ET), padval, jnp.int32)
    return jnp.concatenate([a, pad], axis=1).reshape(_NC * _NS, _NCH, _CP)


_NB = 2
_NBG = 4


def _scatter_body(msg_hbm, idx_hbm, zer_hbm, p_hbm, a_sp, idx_v, bufs,
                  *sems):
    c = lax.axis_index("c")
    s = lax.axis_index("s")
    tid = c * _NS + s
    row0 = tid * _ET
    ls, ss = sems[:_NB], sems[_NB:]
    pltpu.sync_copy(idx_hbm.at[tid], idx_v)
    pltpu.sync_copy(zer_hbm, a_sp.at[pl.ds(s * _ZR, _ZR), :])
    plsc.subcore_barrier()

    def load(j, b):
        return pltpu.async_copy(
            msg_hbm.at[pl.ds(row0 + j * _CP, _CP), :], bufs.at[b], ls[b])

    def drain_scat(b):
        pltpu.make_async_copy(bufs.at[b], a_sp.at[idx_v.at[0]], ss[b]).wait()

    def group(g, carry):
        base = g * _NB
        lds = []
        for b in range(_NB):
            @pl.when(g > 0)
            def _(b=b):
                drain_scat(b)
            lds.append(load(base + b, b))
        for b in range(_NB):
            lds[b].wait()
            pltpu.async_copy(bufs.at[b], a_sp.at[idx_v.at[base + b]], ss[b],
                             add=True)
        return carry

    ngrp = _NF // _NB
    rem = _NF - ngrp * _NB
    lax.fori_loop(0, ngrp, group, 0)
    for b in range(_NB):
        drain_scat(b)
    lds = [load(ngrp * _NB + b, b) for b in range(rem)]
    tl = pltpu.async_copy(msg_hbm.at[pl.ds(row0 + _NF * _CP, _TAIL), :],
                          bufs.at[rem, pl.ds(0, _TAIL), :], ls[rem])
    scs = []
    for b in range(rem):
        lds[b].wait()
        scs.append(pltpu.async_copy(bufs.at[b], a_sp.at[idx_v.at[ngrp * _NB + b]],
                                    ss[b], add=True))
    tl.wait()
    scs.append(pltpu.async_copy(bufs.at[rem], a_sp.at[idx_v.at[_NF]],
                                ss[rem], add=True))
    for d in scs:
        d.wait()
    plsc.subcore_barrier()
    pltpu.sync_copy(a_sp.at[pl.ds(s * _ZR, _ZR), :],
                    p_hbm.at[c, pl.ds(s * _ZR, _ZR), :])


_seg_scatter = pl.kernel(
    _scatter_body,
    out_type=jax.ShapeDtypeStruct((2, _NP, H), jnp.float32),
    mesh=_MESH,
    scratch_types=[
        pltpu.VMEM_SHARED((_NP, H), jnp.float32),
        pltpu.VMEM((_NCH, _CP), jnp.int32),
        pltpu.VMEM((_NB, _CP, H), jnp.float32),
    ] + [pltpu.SemaphoreType.DMA] * (2 * _NB),
)


def _gather_body(tbl_hbm, idx_hbm, g_hbm, idx_v, bufs, *sems):
    c = lax.axis_index("c")
    s = lax.axis_index("s")
    tid = c * _NS + s
    row0 = tid * _ET
    ls, ss = sems[:_NBG], sems[_NBG:]
    pltpu.sync_copy(idx_hbm.at[tid], idx_v)

    def gat(j, b):
        return pltpu.async_copy(tbl_hbm.at[idx_v.at[j]], bufs.at[b], ls[b])

    def stor(j, b):
        return pltpu.async_copy(
            bufs.at[b], g_hbm.at[pl.ds(row0 + j * _CP, _CP), :], ss[b])

    def drain_stor(b):
        pltpu.make_async_copy(
            bufs.at[b], g_hbm.at[pl.ds(row0, _CP), :], ss[b]).wait()

    def group(g, carry):
        base = g * _NBG
        gts = []
        for b in range(_NBG):
            @pl.when(g > 0)
            def _(b=b):
                drain_stor(b)
            gts.append(gat(base + b, b))
        for b in range(_NBG):
            gts[b].wait()
            stor(base + b, b)
        return carry

    ngrp = _NF // _NBG
    rem = _NF - ngrp * _NBG
    lax.fori_loop(0, ngrp, group, 0)
    for b in range(_NBG):
        drain_stor(b)
    gts = [gat(ngrp * _NBG + b, b) for b in range(rem)]
    gt = gat(_NF, rem)
    sts = []
    for b in range(rem):
        gts[b].wait()
        sts.append(stor(ngrp * _NBG + b, b))
    gt.wait()
    sts.append(pltpu.async_copy(
        bufs.at[rem, pl.ds(0, _TAIL), :],
        g_hbm.at[pl.ds(row0 + _NF * _CP, _TAIL), :], ss[rem]))
    for d in sts:
        d.wait()


_seg_gather = pl.kernel(
    _gather_body,
    out_type=jax.ShapeDtypeStruct((E, H), jnp.float32),
    mesh=_MESH,
    scratch_types=[
        pltpu.VMEM((_NCH, _CP), jnp.int32),
        pltpu.VMEM((_NBG, _CP, H), jnp.float32),
    ] + [pltpu.SemaphoreType.DMA] * (2 * _NBG),
)


def _gs_body(tbl_hbm, si_hbm, di_hbm, zer_hbm, p_hbm, a_sp, si_v, di_v, bufs,
             *sems):
    c = lax.axis_index("c")
    s = lax.axis_index("s")
    tid = c * _NS + s
    ls, ss = sems[:_NB], sems[_NB:]
    pltpu.sync_copy(si_hbm.at[tid], si_v)
    pltpu.sync_copy(di_hbm.at[tid], di_v)
    pltpu.sync_copy(zer_hbm, a_sp.at[pl.ds(s * _ZR, _ZR), :])
    plsc.subcore_barrier()

    def drain_scat(b):
        pltpu.make_async_copy(bufs.at[b], a_sp.at[di_v.at[0]], ss[b]).wait()

    def group(g, carry):
        base = g * _NB
        gts = []
        for b in range(_NB):
            @pl.when(g > 0)
            def _(b=b):
                drain_scat(b)
            gts.append(pltpu.async_copy(tbl_hbm.at[si_v.at[base + b]],
                                        bufs.at[b], ls[b]))
        for b in range(_NB):
            gts[b].wait()
            pltpu.async_copy(bufs.at[b], a_sp.at[di_v.at[base + b]], ss[b],
                             add=True)
        return carry

    lax.fori_loop(0, _NCH // _NB, group, 0)
    for b in range(_NB):
        drain_scat(b)
    plsc.subcore_barrier()
    pltpu.sync_copy(a_sp.at[pl.ds(s * _ZR, _ZR), :],
                    p_hbm.at[c, pl.ds(s * _ZR, _ZR), :])


_seg_gather_scatter = pl.kernel(
    _gs_body,
    out_type=jax.ShapeDtypeStruct((2, _NP, H), jnp.float32),
    mesh=_MESH,
    scratch_types=[
        pltpu.VMEM_SHARED((_NP, H), jnp.float32),
        pltpu.VMEM((_NCH, _CP), jnp.int32),
        pltpu.VMEM((_NCH, _CP), jnp.int32),
        pltpu.VMEM((_NB, _CP, H), jnp.float32),
    ] + [pltpu.SemaphoreType.DMA] * (2 * _NB),
)


def _readout_body(hid_hbm, idx_hbm, zer_hbm, one_hbm, sum_hbm, cnt_hbm,
                  s_sp, c_sp, idx_v, buf, obuf):
    c = lax.axis_index("c")
    s = lax.axis_index("s")
    tid = c * _NS + s
    row0 = tid * _AT
    pltpu.sync_copy(idx_hbm.at[tid], idx_v)
    pltpu.sync_copy(one_hbm, obuf)
    pltpu.sync_copy(zer_hbm.at[pl.ds(0, _MT), :],
                    s_sp.at[pl.ds(s * _MT, _MT), :])
    pltpu.sync_copy(zer_hbm.at[pl.ds(0, _MT), :],
                    c_sp.at[pl.ds(s * _MT, _MT), :])
    plsc.subcore_barrier()

    def sbody(j, carry):
        pltpu.sync_copy(hid_hbm.at[pl.ds(row0 + j * _CP, _CP), :], buf)
        pltpu.sync_copy(buf, s_sp.at[idx_v.at[j]], add=True)
        pltpu.sync_copy(obuf, c_sp.at[idx_v.at[j]], add=True)
        return carry

    lax.fori_loop(0, _AT // _CP, sbody, 0)
    plsc.subcore_barrier()
    pltpu.sync_copy(s_sp.at[pl.ds(s * _MT, _MT), :],
                    sum_hbm.at[c, pl.ds(s * _MT, _MT), :])
    pltpu.sync_copy(c_sp.at[pl.ds(s * _MT, _MT), :],
                    cnt_hbm.at[c, pl.ds(s * _MT, _MT), :])


_readout = pl.kernel(
    _readout_body,
    out_type=[jax.ShapeDtypeStruct((2, M, H), jnp.float32),
              jax.ShapeDtypeStruct((2, M, H), jnp.float32)],
    mesh=_MESH,
    scratch_types=[
        pltpu.VMEM_SHARED((M + 8, H), jnp.float32),
        pltpu.VMEM_SHARED((M + 8, H), jnp.float32),
        pltpu.VMEM((_AT // _CP, _CP), jnp.int32),
        pltpu.VMEM((_CP, H), jnp.float32),
        pltpu.VMEM((_CP, H), jnp.float32),
    ],
)


def kernel(f_atoms_r, f_bonds_r, f_atoms_p1, f_bonds_p1, f_atoms_p2,
           f_bonds_p2, W_i, b_i, W_h, b_h, W_o, b_o, Wd_i, bd_i, Wd_h, bd_h,
           Wd_o, bd_o, F1_W, F1_b, F2_W, F2_b, F3_W, F3_b, edge_index_r,
           b2revb_r, edge_index_p1, b2revb_p1, edge_index_p2, b2revb_p2,
           atom2mol, gpu):
    b_i2 = b_i.reshape(1, H)
    b_h2 = b_h.reshape(1, H)
    b_o2 = b_o.reshape(1, H)
    bd_i2 = bd_i.reshape(1, H)
    bd_h2 = bd_h.reshape(1, H)
    bd_o2 = bd_o.reshape(1, H)
    f1b = F1_b.reshape(1, H)
    f2b = F2_b.reshape(1, H)
    f3w = jnp.zeros((H, H), jnp.float32).at[:, :T].set(F3_W)
    f3b = jnp.zeros((1, H), jnp.float32).at[0, :T].set(F3_b)
    wo1, wo2 = W_o[:AF], W_o[AF:]
    wdo1, wdo2 = Wd_o[:H], Wd_o[H:]

    zer = jnp.zeros((_ZR, H), jnp.float32)
    one = jnp.ones((_CP, H), jnp.float32)

    def mpn(f_atoms, f_bonds, ei):
        idx3 = _pad_edge_idx(ei[1], N)
        inp, msg = _bonds_mm(f_bonds, W_i, b_i2)
        for _ in range(DEPTH - 1):
            am = _add2(_seg_scatter(msg, idx3, zer))
            g = _seg_gather(am, idx3)
            msg = _iter_mm(g, msg, W_h, inp, b_h2)
        am = _seg_scatter(msg, idx3, zer)
        return _out_mm(f_atoms, am, wo1, wo2, b_o2)

    r_atoms = mpn(f_atoms_r, f_bonds_r, edge_index_r)
    p1_atoms = mpn(f_atoms_p1, f_bonds_p1, edge_index_p1)
    p2_atoms = mpn(f_atoms_p2, f_bonds_p2, edge_index_p2)

    a2mp = jnp.concatenate(
        [atom2mol.astype(jnp.int32),
         jnp.full((_NHP - N,), M, jnp.int32)]).reshape(_NC * _NS,
                                                       _AT // _CP, _CP)

    def diff_head(ap, ei):
        si = _pad_edge_idx(ei[0], 0)
        di = _pad_edge_idx(ei[1], N)
        d, inp, msg = _diff_mm(ap, r_atoms, Wd_i, bd_i2)
        for _ in range(DEPTH - 1):
            agg = _seg_gather_scatter(msg, si, di, zer)
            msg = _diter_mm(agg, Wd_h, inp, bd_h2)
        hid = _hid_mm(d, msg, wdo1, wdo2, bd_o2, _NHP)
        return _readout(hid, a2mp, zer, one)

    s1, c1 = diff_head(p1_atoms, edge_index_p1)
    s2, _ = diff_head(p2_atoms, edge_index_p2)
    out = _ffn(s1, s2, c1, F1_W, f1b, F2_W, f2b, f3w, f3b)
    return out[:, :T]


# speedup vs baseline: 1.0181x; 1.0181x over previous
import jax
import jax.numpy as jnp
from jax import lax
from jax.experimental import pallas as pl
from jax.experimental.pallas import tpu as pltpu
from jax.experimental.pallas import tpu_sc as plsc

N = 10000
E2 = 80000
E = 160000
AF = 133
BFD = 147
H = 128
M = 512
T = 2
DEPTH = 3

_INTERP = False


def _bonds_body(x_ref, w_ref, b_ref, inp_ref, msg_ref):
    t = jnp.dot(x_ref[...], w_ref[...], preferred_element_type=jnp.float32)
    t = t + b_ref[...]
    inp_ref[...] = t
    msg_ref[...] = jnp.maximum(t, 0.0)


def _bonds_mm(x, w, b):
    blk = 2000
    r = x.shape[0]
    return pl.pallas_call(
        _bonds_body,
        grid=(r // blk,),
        in_specs=[
            pl.BlockSpec((blk, BFD), lambda i: (i, 0)),
            pl.BlockSpec((BFD, H), lambda i: (0, 0)),
            pl.BlockSpec((1, H), lambda i: (0, 0)),
        ],
        out_specs=[
            pl.BlockSpec((blk, H), lambda i: (i, 0)),
            pl.BlockSpec((blk, H), lambda i: (i, 0)),
        ],
        out_shape=[jax.ShapeDtypeStruct((r, H), jnp.float32)] * 2,
        interpret=_INTERP,
    )(x, w, b)


def _add2_body(p_ref, o_ref):
    o_ref[...] = p_ref[0] + p_ref[1]


def _add2(p):
    rows = p.shape[1]
    blk = 5056
    return pl.pallas_call(
        _add2_body,
        grid=(rows // blk,),
        in_specs=[pl.BlockSpec((2, blk, H), lambda i: (0, i, 0))],
        out_specs=pl.BlockSpec((blk, H), lambda i: (i, 0)),
        out_shape=jax.ShapeDtypeStruct((rows, H), jnp.float32),
        interpret=_INTERP,
    )(p)


def _iter_body(g_ref, m_ref, w_ref, inp_ref, b_ref, o_ref):
    t = g_ref[...] - m_ref[...]
    u = jnp.dot(t, w_ref[...], preferred_element_type=jnp.float32)
    o_ref[...] = jnp.maximum(inp_ref[...] + u + b_ref[...], 0.0)


def _iter_mm(g, msg, w, inp, b):
    blk = 3200
    nblk = E // blk
    half = E2 // blk
    sw = lambda i: ((i + half) % nblk, 0)
    return pl.pallas_call(
        _iter_body,
        grid=(nblk,),
        in_specs=[
            pl.BlockSpec((blk, H), lambda i: (i, 0)),
            pl.BlockSpec((blk, H), lambda i: (i, 0)),
            pl.BlockSpec((H, H), lambda i: (0, 0)),
            pl.BlockSpec((blk, H), sw),
            pl.BlockSpec((1, H), lambda i: (0, 0)),
        ],
        out_specs=pl.BlockSpec((blk, H), sw),
        out_shape=jax.ShapeDtypeStruct((E, H), jnp.float32),
        interpret=_INTERP,
    )(g, msg, w, inp, b)


def _out_body(fa_ref, am_ref, w1_ref, w2_ref, b_ref, o_ref):
    t = jnp.dot(fa_ref[...], w1_ref[...], preferred_element_type=jnp.float32)
    am = am_ref[0] + am_ref[1]
    t = t + jnp.dot(am, w2_ref[...], preferred_element_type=jnp.float32)
    o_ref[...] = jnp.maximum(t + b_ref[...], 0.0)


def _out_mm(fa, am, w1, w2, b):
    blk = 2000
    return pl.pallas_call(
        _out_body,
        grid=(N // blk,),
        in_specs=[
            pl.BlockSpec((blk, AF), lambda i: (i, 0)),
            pl.BlockSpec((2, blk, H), lambda i: (0, i, 0)),
            pl.BlockSpec((AF, H), lambda i: (0, 0)),
            pl.BlockSpec((H, H), lambda i: (0, 0)),
            pl.BlockSpec((1, H), lambda i: (0, 0)),
        ],
        out_specs=pl.BlockSpec((blk, H), lambda i: (i, 0)),
        out_shape=jax.ShapeDtypeStruct((N, H), jnp.float32),
        interpret=_INTERP,
    )(fa, am, w1, w2, b)


def _diff_body(ap_ref, ar_ref, w_ref, b_ref, d_ref, inp_ref, m_ref):
    d = ap_ref[...] - ar_ref[...]
    t = jnp.dot(d, w_ref[...], preferred_element_type=jnp.float32) + b_ref[...]
    d_ref[...] = d
    inp_ref[...] = t
    m_ref[...] = jnp.maximum(t, 0.0)


def _diff_mm(ap, ar, w, b):
    blk = 2000
    return pl.pallas_call(
        _diff_body,
        grid=(N // blk,),
        in_specs=[
            pl.BlockSpec((blk, H), lambda i: (i, 0)),
            pl.BlockSpec((blk, H), lambda i: (i, 0)),
            pl.BlockSpec((H, H), lambda i: (0, 0)),
            pl.BlockSpec((1, H), lambda i: (0, 0)),
        ],
        out_specs=[
            pl.BlockSpec((blk, H), lambda i: (i, 0)),
            pl.BlockSpec((blk, H), lambda i: (i, 0)),
            pl.BlockSpec((blk, H), lambda i: (i, 0)),
        ],
        out_shape=[jax.ShapeDtypeStruct((N, H), jnp.float32)] * 3,
        interpret=_INTERP,
    )(ap, ar, w, b)


def _diter_body(agg_ref, w_ref, inp_ref, b_ref, m_ref):
    a = agg_ref[0] + agg_ref[1]
    u = jnp.dot(a, w_ref[...], preferred_element_type=jnp.float32)
    m_ref[...] = jnp.maximum(inp_ref[...] + u + b_ref[...], 0.0)


def _diter_mm(agg, w, inp, b):
    blk = 2000
    return pl.pallas_call(
        _diter_body,
        grid=(N // blk,),
        in_specs=[
            pl.BlockSpec((2, blk, H), lambda i: (0, i, 0)),
            pl.BlockSpec((H, H), lambda i: (0, 0)),
            pl.BlockSpec((blk, H), lambda i: (i, 0)),
            pl.BlockSpec((1, H), lambda i: (0, 0)),
        ],
        out_specs=pl.BlockSpec((blk, H), lambda i: (i, 0)),
        out_shape=jax.ShapeDtypeStruct((N, H), jnp.float32),
        interpret=_INTERP,
    )(agg, w, inp, b)


def _hid_body(d_ref, m_ref, w1_ref, w2_ref, b_ref, h_ref):
    t = jnp.dot(d_ref[...], w1_ref[...], preferred_element_type=jnp.float32)
    t = t + jnp.dot(m_ref[...], w2_ref[...], preferred_element_type=jnp.float32)
    h_ref[...] = jnp.maximum(t + b_ref[...], 0.0)


def _hid_mm(d, msg, w1, w2, b, rpad):
    blk = 2000
    return pl.pallas_call(
        _hid_body,
        grid=(N // blk,),
        in_specs=[
            pl.BlockSpec((blk, H), lambda i: (i, 0)),
            pl.BlockSpec((blk, H), lambda i: (i, 0)),
            pl.BlockSpec((H, H), lambda i: (0, 0)),
            pl.BlockSpec((H, H), lambda i: (0, 0)),
            pl.BlockSpec((1, H), lambda i: (0, 0)),
        ],
        out_specs=pl.BlockSpec((blk, H), lambda i: (i, 0)),
        out_shape=jax.ShapeDtypeStruct((rpad, H), jnp.float32),
        interpret=_INTERP,
    )(d, msg, w1, w2, b)


def _ffn_body(s1_ref, s2_ref, c_ref, w1_ref, b1_ref, w2_ref, b2_ref,
              w3_ref, b3_ref, o_ref):
    r = 1.0 / jnp.maximum(c_ref[0] + c_ref[1], 1.0)

    def head(s_ref):
        m = (s_ref[0] + s_ref[1]) * r
        h1 = jnp.dot(m, w1_ref[...], preferred_element_type=jnp.float32)
        h1 = jnp.maximum(h1 + b1_ref[...], 0.0)
        h2 = jnp.dot(h1, w2_ref[...], preferred_element_type=jnp.float32)
        h2 = jnp.maximum(h2 + b2_ref[...], 0.0)
        h3 = jnp.dot(h2, w3_ref[...], preferred_element_type=jnp.float32)
        return jnp.clip(h3 + b3_ref[...], 0.0, 6.0)

    o_ref[...] = jax.nn.sigmoid(head(s1_ref) - head(s2_ref))


def _ffn(s1, s2, cnt, w1, b1, w2, b2, w3, b3):
    full = lambda a, b: pl.BlockSpec((a, b), lambda: (0, 0))
    st = pl.BlockSpec((2, M, H), lambda: (0, 0, 0))
    return pl.pallas_call(
        _ffn_body,
        in_specs=[
            st, st, st,
            full(H, H), full(1, H), full(H, H), full(1, H),
            full(H, H), full(1, H),
        ],
        out_specs=full(M, H),
        out_shape=jax.ShapeDtypeStruct((M, H), jnp.float32),
        interpret=_INTERP,
    )(s1, s2, cnt, w1, b1, w2, b2, w3, b3)


_NC = 2
_NS = 16
_CP = 128
_ET = E // (_NC * _NS)
_NF = _ET // _CP
_TAIL = _ET - _NF * _CP
_NCH = _NF + 1
_ZR = 632
_NP = _NS * _ZR
_AT = 384
_NHP = _NC * _NS * _AT
_MT = M // _NS

_MESH = plsc.VectorSubcoreMesh(
    core_axis_name="c", subcore_axis_name="s",
    num_cores=_NC, num_subcores=_NS)


def _pad_edge_idx(idx, padval):
    a = idx.astype(jnp.int32).reshape(_NC * _NS, _ET)
    pad = jnp.full((_NC * _NS, _NCH * _CP - _ET), padval, jnp.int32)
    return jnp.concatenate([a, pad], axis=1).reshape(_NC * _NS, _NCH, _CP)


_NB = 2
_NBG = 4


def _scatter_body(msg_hbm, idx_hbm, zer_hbm, p_hbm, a_sp, idx_v, bufs,
                  *sems):
    c = lax.axis_index("c")
    s = lax.axis_index("s")
    tid = c * _NS + s
    row0 = tid * _ET
    ls, ss = sems[:_NB], sems[_NB:]
    pltpu.sync_copy(idx_hbm.at[tid], idx_v)
    pltpu.sync_copy(zer_hbm, a_sp.at[pl.ds(s * _ZR, _ZR), :])
    plsc.subcore_barrier()

    def load(j, b):
        return pltpu.async_copy(
            msg_hbm.at[pl.ds(row0 + j * _CP, _CP), :], bufs.at[b], ls[b])

    def drain_scat(b):
        pltpu.make_async_copy(bufs.at[b], a_sp.at[idx_v.at[0]], ss[b]).wait()

    def group(g, carry):
        base = g * _NB
        lds = []
        for b in range(_NB):
            @pl.when(g > 0)
            def _(b=b):
                drain_scat(b)
            lds.append(load(base + b, b))
        for b in range(_NB):
            lds[b].wait()
            pltpu.async_copy(bufs.at[b], a_sp.at[idx_v.at[base + b]], ss[b],
                             add=True)
        return carry

    ngrp = _NF // _NB
    rem = _NF - ngrp * _NB
    lax.fori_loop(0, ngrp, group, 0)
    for b in range(_NB):
        drain_scat(b)
    lds = [load(ngrp * _NB + b, b) for b in range(rem)]
    tl = pltpu.async_copy(msg_hbm.at[pl.ds(row0 + _NF * _CP, _TAIL), :],
                          bufs.at[rem, pl.ds(0, _TAIL), :], ls[rem])
    scs = []
    for b in range(rem):
        lds[b].wait()
        scs.append(pltpu.async_copy(bufs.at[b], a_sp.at[idx_v.at[ngrp * _NB + b]],
                                    ss[b], add=True))
    tl.wait()
    scs.append(pltpu.async_copy(bufs.at[rem], a_sp.at[idx_v.at[_NF]],
                                ss[rem], add=True))
    for d in scs:
        d.wait()
    plsc.subcore_barrier()
    pltpu.sync_copy(a_sp.at[pl.ds(s * _ZR, _ZR), :],
                    p_hbm.at[c, pl.ds(s * _ZR, _ZR), :])


_seg_scatter = pl.kernel(
    _scatter_body,
    out_type=jax.ShapeDtypeStruct((2, _NP, H), jnp.float32),
    mesh=_MESH,
    scratch_types=[
        pltpu.VMEM_SHARED((_NP, H), jnp.float32),
        pltpu.VMEM((_NCH, _CP), jnp.int32),
        pltpu.VMEM((_NB, _CP, H), jnp.float32),
    ] + [pltpu.SemaphoreType.DMA] * (2 * _NB),
)


def _gather_body(tbl_hbm, idx_hbm, g_hbm, idx_v, bufs, *sems):
    c = lax.axis_index("c")
    s = lax.axis_index("s")
    tid = c * _NS + s
    row0 = tid * _ET
    ls, ss = sems[:_NBG], sems[_NBG:]
    pltpu.sync_copy(idx_hbm.at[tid], idx_v)

    def gat(j, b):
        return pltpu.async_copy(tbl_hbm.at[idx_v.at[j]], bufs.at[b], ls[b])

    def stor(j, b):
        return pltpu.async_copy(
            bufs.at[b], g_hbm.at[pl.ds(row0 + j * _CP, _CP), :], ss[b])

    def drain_stor(b):
        pltpu.make_async_copy(
            bufs.at[b], g_hbm.at[pl.ds(row0, _CP), :], ss[b]).wait()

    def group(g, carry):
        base = g * _NBG
        gts = []
        for b in range(_NBG):
            @pl.when(g > 0)
            def _(b=b):
                drain_stor(b)
            gts.append(gat(base + b, b))
        for b in range(_NBG):
            gts[b].wait()
            stor(base + b, b)
        return carry

    ngrp = _NF // _NBG
    rem = _NF - ngrp * _NBG
    lax.fori_loop(0, ngrp, group, 0)
    for b in range(_NBG):
        drain_stor(b)
    gts = [gat(ngrp * _NBG + b, b) for b in range(rem)]
    gt = gat(_NF, rem)
    sts = []
    for b in range(rem):
        gts[b].wait()
        sts.append(stor(ngrp * _NBG + b, b))
    gt.wait()
    sts.append(pltpu.async_copy(
        bufs.at[rem, pl.ds(0, _TAIL), :],
        g_hbm.at[pl.ds(row0 + _NF * _CP, _TAIL), :], ss[rem]))
    for d in sts:
        d.wait()


_seg_gather = pl.kernel(
    _gather_body,
    out_type=jax.ShapeDtypeStruct((E, H), jnp.float32),
    mesh=_MESH,
    scratch_types=[
        pltpu.VMEM((_NCH, _CP), jnp.int32),
        pltpu.VMEM((_NBG, _CP, H), jnp.float32),
    ] + [pltpu.SemaphoreType.DMA] * (2 * _NBG),
)


def _gs_body(tbl_hbm, si_hbm, di_hbm, zer_hbm, p_hbm, a_sp, si_v, di_v, bufs,
             *sems):
    c = lax.axis_index("c")
    s = lax.axis_index("s")
    tid = c * _NS + s
    ls, ss = sems[:_NB], sems[_NB:]
    pltpu.sync_copy(si_hbm.at[tid], si_v)
    pltpu.sync_copy(di_hbm.at[tid], di_v)
    pltpu.sync_copy(zer_hbm, a_sp.at[pl.ds(s * _ZR, _ZR), :])
    plsc.subcore_barrier()

    def drain_scat(b):
        pltpu.make_async_copy(bufs.at[b], a_sp.at[di_v.at[0]], ss[b]).wait()

    def group(g, carry):
        base = g * _NB
        gts = []
        for b in range(_NB):
            @pl.when(g > 0)
            def _(b=b):
                drain_scat(b)
            gts.append(pltpu.async_copy(tbl_hbm.at[si_v.at[base + b]],
                                        bufs.at[b], ls[b]))
        for b in range(_NB):
            gts[b].wait()
            pltpu.async_copy(bufs.at[b], a_sp.at[di_v.at[base + b]], ss[b],
                             add=True)
        return carry

    lax.fori_loop(0, _NCH // _NB, group, 0)
    for b in range(_NB):
        drain_scat(b)
    plsc.subcore_barrier()
    pltpu.sync_copy(a_sp.at[pl.ds(s * _ZR, _ZR), :],
                    p_hbm.at[c, pl.ds(s * _ZR, _ZR), :])


_seg_gather_scatter = pl.kernel(
    _gs_body,
    out_type=jax.ShapeDtypeStruct((2, _NP, H), jnp.float32),
    mesh=_MESH,
    scratch_types=[
        pltpu.VMEM_SHARED((_NP, H), jnp.float32),
        pltpu.VMEM((_NCH, _CP), jnp.int32),
        pltpu.VMEM((_NCH, _CP), jnp.int32),
        pltpu.VMEM((_NB, _CP, H), jnp.float32),
    ] + [pltpu.SemaphoreType.DMA] * (2 * _NB),
)


def _readout_body(hid_hbm, idx_hbm, zer_hbm, one_hbm, sum_hbm, cnt_hbm,
                  s_sp, c_sp, idx_v, buf, obuf):
    c = lax.axis_index("c")
    s = lax.axis_index("s")
    tid = c * _NS + s
    row0 = tid * _AT
    pltpu.sync_copy(idx_hbm.at[tid], idx_v)
    pltpu.sync_copy(one_hbm, obuf)
    pltpu.sync_copy(zer_hbm.at[pl.ds(0, _MT), :],
                    s_sp.at[pl.ds(s * _MT, _MT), :])
    pltpu.sync_copy(zer_hbm.at[pl.ds(0, _MT), :],
                    c_sp.at[pl.ds(s * _MT, _MT), :])
    plsc.subcore_barrier()

    def sbody(j, carry):
        pltpu.sync_copy(hid_hbm.at[pl.ds(row0 + j * _CP, _CP), :], buf)
        pltpu.sync_copy(buf, s_sp.at[idx_v.at[j]], add=True)
        pltpu.sync_copy(obuf, c_sp.at[idx_v.at[j]], add=True)
        return carry

    lax.fori_loop(0, _AT // _CP, sbody, 0)
    plsc.subcore_barrier()
    pltpu.sync_copy(s_sp.at[pl.ds(s * _MT, _MT), :],
                    sum_hbm.at[c, pl.ds(s * _MT, _MT), :])
    pltpu.sync_copy(c_sp.at[pl.ds(s * _MT, _MT), :],
                    cnt_hbm.at[c, pl.ds(s * _MT, _MT), :])


_readout = pl.kernel(
    _readout_body,
    out_type=[jax.ShapeDtypeStruct((2, M, H), jnp.float32),
              jax.ShapeDtypeStruct((2, M, H), jnp.float32)],
    mesh=_MESH,
    scratch_types=[
        pltpu.VMEM_SHARED((M + 8, H), jnp.float32),
        pltpu.VMEM_SHARED((M + 8, H), jnp.float32),
        pltpu.VMEM((_AT // _CP, _CP), jnp.int32),
        pltpu.VMEM((_CP, H), jnp.float32),
        pltpu.VMEM((_CP, H), jnp.float32),
    ],
)


def kernel(f_atoms_r, f_bonds_r, f_atoms_p1, f_bonds_p1, f_atoms_p2,
           f_bonds_p2, W_i, b_i, W_h, b_h, W_o, b_o, Wd_i, bd_i, Wd_h, bd_h,
           Wd_o, bd_o, F1_W, F1_b, F2_W, F2_b, F3_W, F3_b, edge_index_r,
           b2revb_r, edge_index_p1, b2revb_p1, edge_index_p2, b2revb_p2,
           atom2mol, gpu):
    b_i2 = b_i.reshape(1, H)
    b_h2 = b_h.reshape(1, H)
    b_o2 = b_o.reshape(1, H)
    bd_i2 = bd_i.reshape(1, H)
    bd_h2 = bd_h.reshape(1, H)
    bd_o2 = bd_o.reshape(1, H)
    f1b = F1_b.reshape(1, H)
    f2b = F2_b.reshape(1, H)
    f3w = jnp.zeros((H, H), jnp.float32).at[:, :T].set(F3_W)
    f3b = jnp.zeros((1, H), jnp.float32).at[0, :T].set(F3_b)
    wo1, wo2 = W_o[:AF], W_o[AF:]
    wdo1, wdo2 = Wd_o[:H], Wd_o[H:]

    zer = jnp.zeros((_ZR, H), jnp.float32)
    one = jnp.ones((_CP, H), jnp.float32)

    def mpn(f_atoms, f_bonds, ei):
        idx3 = _pad_edge_idx(ei[1], N)
        inp, msg = _bonds_mm(f_bonds, W_i, b_i2)
        for _ in range(DEPTH - 1):
            am = _add2(_seg_scatter(msg, idx3, zer))
            g = _seg_gather(am, idx3)
            msg = _iter_mm(g, msg, W_h, inp, b_h2)
        am = _seg_scatter(msg, idx3, zer)
        return _out_mm(f_atoms, am, wo1, wo2, b_o2)

    r_atoms = mpn(f_atoms_r, f_bonds_r, edge_index_r)
    p1_atoms = mpn(f_atoms_p1, f_bonds_p1, edge_index_p1)
    p2_atoms = mpn(f_atoms_p2, f_bonds_p2, edge_index_p2)

    a2mp = jnp.concatenate(
        [atom2mol.astype(jnp.int32),
         jnp.full((_NHP - N,), M, jnp.int32)]).reshape(_NC * _NS,
                                                       _AT // _CP, _CP)

    def diff_head(ap, ei):
        si = _pad_edge_idx(ei[0], 0)
        di = _pad_edge_idx(ei[1], N)
        d, inp, msg = _diff_mm(ap, r_atoms, Wd_i, bd_i2)
        for _ in range(DEPTH - 1):
            agg = _seg_gather_scatter(msg, si, di, zer)
            msg = _diter_mm(agg, Wd_h, inp, bd_h2)
        hid = _hid_mm(d, msg, wdo1, wdo2, bd_o2, _NHP)
        return _readout(hid, a2mp, zer, one)

    s1, c1 = diff_head(p1_atoms, edge_index_p1)
    s2, _ = diff_head(p2_atoms, edge_index_p2)
    out = _ffn(s1, s2, c1, F1_W, f1b, F2_W, f2b, f3w, f3b)
    return out[:, :T]


# speedup vs baseline: 1.3865x; 1.3618x over previous
import jax
import jax.numpy as jnp
from jax import lax
from jax.experimental import pallas as pl
from jax.experimental.pallas import tpu as pltpu
from jax.experimental.pallas import tpu_sc as plsc

N = 10000
E2 = 80000
E = 160000
AF = 133
BFD = 147
H = 128
M = 512
T = 2
DEPTH = 3

_INTERP = False


def _bonds_body(x_ref, w_ref, b_ref, inp_ref, msg_ref):
    t = jnp.dot(x_ref[...], w_ref[...], preferred_element_type=jnp.float32)
    t = t + b_ref[...]
    inp_ref[...] = t
    msg_ref[...] = jnp.maximum(t, 0.0)


def _bonds_mm(x, w, b):
    blk = 2000
    r = x.shape[0]
    return pl.pallas_call(
        _bonds_body,
        grid=(r // blk,),
        in_specs=[
            pl.BlockSpec((blk, BFD), lambda i: (i, 0)),
            pl.BlockSpec((BFD, H), lambda i: (0, 0)),
            pl.BlockSpec((1, H), lambda i: (0, 0)),
        ],
        out_specs=[
            pl.BlockSpec((blk, H), lambda i: (i, 0)),
            pl.BlockSpec((blk, H), lambda i: (i, 0)),
        ],
        out_shape=[jax.ShapeDtypeStruct((r, H), jnp.float32)] * 2,
        interpret=_INTERP,
    )(x, w, b)


def _add2_body(p_ref, o_ref):
    o_ref[...] = p_ref[0] + p_ref[1]


def _add2(p):
    rows = p.shape[1]
    blk = 5056
    return pl.pallas_call(
        _add2_body,
        grid=(rows // blk,),
        in_specs=[pl.BlockSpec((2, blk, H), lambda i: (0, i, 0))],
        out_specs=pl.BlockSpec((blk, H), lambda i: (i, 0)),
        out_shape=jax.ShapeDtypeStruct((rows, H), jnp.float32),
        interpret=_INTERP,
    )(p)


def _iter_body(g_ref, m_ref, w_ref, inp_ref, b_ref, o_ref):
    t = (g_ref[0] + g_ref[1]) - m_ref[...]
    u = jnp.dot(t, w_ref[...], preferred_element_type=jnp.float32)
    o_ref[...] = jnp.maximum(inp_ref[...] + u + b_ref[...], 0.0)


def _iter_mm(g, msg, w, inp, b):
    blk = 3200
    nblk = E // blk
    half = E2 // blk
    sw = lambda i: ((i + half) % nblk, 0)
    return pl.pallas_call(
        _iter_body,
        grid=(nblk,),
        in_specs=[
            pl.BlockSpec((2, blk, H), lambda i: (0, i, 0)),
            pl.BlockSpec((blk, H), lambda i: (i, 0)),
            pl.BlockSpec((H, H), lambda i: (0, 0)),
            pl.BlockSpec((blk, H), sw),
            pl.BlockSpec((1, H), lambda i: (0, 0)),
        ],
        out_specs=pl.BlockSpec((blk, H), sw),
        out_shape=jax.ShapeDtypeStruct((E, H), jnp.float32),
        interpret=_INTERP,
    )(g, msg, w, inp, b)


def _out_body(fa_ref, am_ref, w1_ref, w2_ref, b_ref, o_ref):
    t = jnp.dot(fa_ref[...], w1_ref[...], preferred_element_type=jnp.float32)
    am = am_ref[0] + am_ref[1]
    t = t + jnp.dot(am, w2_ref[...], preferred_element_type=jnp.float32)
    o_ref[...] = jnp.maximum(t + b_ref[...], 0.0)


def _out_mm(fa, am, w1, w2, b):
    blk = 2000
    return pl.pallas_call(
        _out_body,
        grid=(N // blk,),
        in_specs=[
            pl.BlockSpec((blk, AF), lambda i: (i, 0)),
            pl.BlockSpec((2, blk, H), lambda i: (0, i, 0)),
            pl.BlockSpec((AF, H), lambda i: (0, 0)),
            pl.BlockSpec((H, H), lambda i: (0, 0)),
            pl.BlockSpec((1, H), lambda i: (0, 0)),
        ],
        out_specs=pl.BlockSpec((blk, H), lambda i: (i, 0)),
        out_shape=jax.ShapeDtypeStruct((N, H), jnp.float32),
        interpret=_INTERP,
    )(fa, am, w1, w2, b)


def _diff_body(ap_ref, ar_ref, w_ref, b_ref, d_ref, inp_ref, m_ref):
    d = ap_ref[...] - ar_ref[...]
    t = jnp.dot(d, w_ref[...], preferred_element_type=jnp.float32) + b_ref[...]
    d_ref[...] = d
    inp_ref[...] = t
    m_ref[...] = jnp.maximum(t, 0.0)


def _diff_mm(ap, ar, w, b):
    blk = 2000
    return pl.pallas_call(
        _diff_body,
        grid=(N // blk,),
        in_specs=[
            pl.BlockSpec((blk, H), lambda i: (i, 0)),
            pl.BlockSpec((blk, H), lambda i: (i, 0)),
            pl.BlockSpec((H, H), lambda i: (0, 0)),
            pl.BlockSpec((1, H), lambda i: (0, 0)),
        ],
        out_specs=[
            pl.BlockSpec((blk, H), lambda i: (i, 0)),
            pl.BlockSpec((blk, H), lambda i: (i, 0)),
            pl.BlockSpec((blk, H), lambda i: (i, 0)),
        ],
        out_shape=[jax.ShapeDtypeStruct((N, H), jnp.float32)] * 3,
        interpret=_INTERP,
    )(ap, ar, w, b)


def _diter_body(agg_ref, w_ref, inp_ref, b_ref, m_ref):
    a = agg_ref[0] + agg_ref[1]
    u = jnp.dot(a, w_ref[...], preferred_element_type=jnp.float32)
    m_ref[...] = jnp.maximum(inp_ref[...] + u + b_ref[...], 0.0)


def _diter_mm(agg, w, inp, b):
    blk = 2000
    return pl.pallas_call(
        _diter_body,
        grid=(N // blk,),
        in_specs=[
            pl.BlockSpec((2, blk, H), lambda i: (0, i, 0)),
            pl.BlockSpec((H, H), lambda i: (0, 0)),
            pl.BlockSpec((blk, H), lambda i: (i, 0)),
            pl.BlockSpec((1, H), lambda i: (0, 0)),
        ],
        out_specs=pl.BlockSpec((blk, H), lambda i: (i, 0)),
        out_shape=jax.ShapeDtypeStruct((N, H), jnp.float32),
        interpret=_INTERP,
    )(agg, w, inp, b)


def _hid_body(d_ref, m_ref, w1_ref, w2_ref, b_ref, h_ref):
    t = jnp.dot(d_ref[...], w1_ref[...], preferred_element_type=jnp.float32)
    t = t + jnp.dot(m_ref[...], w2_ref[...], preferred_element_type=jnp.float32)
    h_ref[...] = jnp.maximum(t + b_ref[...], 0.0)


def _hid_mm(d, msg, w1, w2, b, rpad):
    blk = 2000
    return pl.pallas_call(
        _hid_body,
        grid=(N // blk,),
        in_specs=[
            pl.BlockSpec((blk, H), lambda i: (i, 0)),
            pl.BlockSpec((blk, H), lambda i: (i, 0)),
            pl.BlockSpec((H, H), lambda i: (0, 0)),
            pl.BlockSpec((H, H), lambda i: (0, 0)),
            pl.BlockSpec((1, H), lambda i: (0, 0)),
        ],
        out_specs=pl.BlockSpec((blk, H), lambda i: (i, 0)),
        out_shape=jax.ShapeDtypeStruct((rpad, H), jnp.float32),
        interpret=_INTERP,
    )(d, msg, w1, w2, b)


def _ffn_body(s1_ref, s2_ref, c_ref, w1_ref, b1_ref, w2_ref, b2_ref,
              w3_ref, b3_ref, o_ref):
    r = 1.0 / jnp.maximum(c_ref[0] + c_ref[1], 1.0)

    def head(s_ref):
        m = (s_ref[0] + s_ref[1]) * r
        h1 = jnp.dot(m, w1_ref[...], preferred_element_type=jnp.float32)
        h1 = jnp.maximum(h1 + b1_ref[...], 0.0)
        h2 = jnp.dot(h1, w2_ref[...], preferred_element_type=jnp.float32)
        h2 = jnp.maximum(h2 + b2_ref[...], 0.0)
        h3 = jnp.dot(h2, w3_ref[...], preferred_element_type=jnp.float32)
        return jnp.clip(h3 + b3_ref[...], 0.0, 6.0)

    o_ref[...] = jax.nn.sigmoid(head(s1_ref) - head(s2_ref))


def _ffn(s1, s2, cnt, w1, b1, w2, b2, w3, b3):
    full = lambda a, b: pl.BlockSpec((a, b), lambda: (0, 0))
    st = pl.BlockSpec((2, M, H), lambda: (0, 0, 0))
    return pl.pallas_call(
        _ffn_body,
        in_specs=[
            st, st, st,
            full(H, H), full(1, H), full(H, H), full(1, H),
            full(H, H), full(1, H),
        ],
        out_specs=full(M, H),
        out_shape=jax.ShapeDtypeStruct((M, H), jnp.float32),
        interpret=_INTERP,
    )(s1, s2, cnt, w1, b1, w2, b2, w3, b3)


_NC = 2
_NS = 16
_CP = 128
_ET = E // (_NC * _NS)
_NF = _ET // _CP
_TAIL = _ET - _NF * _CP
_NCH = _NF + 1
_ZR = 632
_NP = _NS * _ZR
_AT = 384
_NHP = _NC * _NS * _AT
_MT = M // _NS

_MESH = plsc.VectorSubcoreMesh(
    core_axis_name="c", subcore_axis_name="s",
    num_cores=_NC, num_subcores=_NS)


def _pad_edge_idx(idx, padval):
    a = idx.astype(jnp.int32).reshape(_NC * _NS, _ET)
    pad = jnp.full((_NC * _NS, _NCH * _CP - _ET), padval, jnp.int32)
    return jnp.concatenate([a, pad], axis=1).reshape(_NC * _NS, _NCH, _CP)


_NB = 2
_NBG = 4


def _scatter_body(msg_hbm, idx_hbm, zer_hbm, p_hbm, a_sp, idx_v, bufs,
                  *sems):
    c = lax.axis_index("c")
    s = lax.axis_index("s")
    tid = c * _NS + s
    row0 = tid * _ET
    ls, ss = sems[:_NB], sems[_NB:]
    pltpu.sync_copy(idx_hbm.at[tid], idx_v)
    pltpu.sync_copy(zer_hbm, a_sp.at[pl.ds(s * _ZR, _ZR), :])
    plsc.subcore_barrier()

    def load(j, b):
        return pltpu.async_copy(
            msg_hbm.at[pl.ds(row0 + j * _CP, _CP), :], bufs.at[b], ls[b])

    def drain_scat(b):
        pltpu.make_async_copy(bufs.at[b], a_sp.at[idx_v.at[0]], ss[b]).wait()

    def group(g, carry):
        base = g * _NB
        lds = []
        for b in range(_NB):
            @pl.when(g > 0)
            def _(b=b):
                drain_scat(b)
            lds.append(load(base + b, b))
        for b in range(_NB):
            lds[b].wait()
            pltpu.async_copy(bufs.at[b], a_sp.at[idx_v.at[base + b]], ss[b],
                             add=True)
        return carry

    ngrp = _NF // _NB
    rem = _NF - ngrp * _NB
    lax.fori_loop(0, ngrp, group, 0)
    for b in range(_NB):
        drain_scat(b)
    lds = [load(ngrp * _NB + b, b) for b in range(rem)]
    tl = pltpu.async_copy(msg_hbm.at[pl.ds(row0 + _NF * _CP, _TAIL), :],
                          bufs.at[rem, pl.ds(0, _TAIL), :], ls[rem])
    scs = []
    for b in range(rem):
        lds[b].wait()
        scs.append(pltpu.async_copy(bufs.at[b], a_sp.at[idx_v.at[ngrp * _NB + b]],
                                    ss[b], add=True))
    tl.wait()
    scs.append(pltpu.async_copy(bufs.at[rem], a_sp.at[idx_v.at[_NF]],
                                ss[rem], add=True))
    for d in scs:
        d.wait()
    plsc.subcore_barrier()
    pltpu.sync_copy(a_sp.at[pl.ds(s * _ZR, _ZR), :],
                    p_hbm.at[c, pl.ds(s * _ZR, _ZR), :])


_seg_scatter = pl.kernel(
    _scatter_body,
    out_type=jax.ShapeDtypeStruct((2, _NP, H), jnp.float32),
    mesh=_MESH,
    scratch_types=[
        pltpu.VMEM_SHARED((_NP, H), jnp.float32),
        pltpu.VMEM((_NCH, _CP), jnp.int32),
        pltpu.VMEM((_NB, _CP, H), jnp.float32),
    ] + [pltpu.SemaphoreType.DMA] * (2 * _NB),
)


def _sgf_body(msg_hbm, idx_hbm, zer_hbm, g_hbm, a_sp, idx_v, gidx_v, bufs,
              *sems):
    c = lax.axis_index("c")
    s = lax.axis_index("s")
    tid = c * _NS + s
    row0 = tid * _ET
    ls, ss = sems[:_NB], sems[_NB:]
    pltpu.sync_copy(idx_hbm.at[tid], idx_v)
    pltpu.sync_copy(idx_hbm.at[2 * s], gidx_v.at[0])
    pltpu.sync_copy(idx_hbm.at[2 * s + 1], gidx_v.at[1])
    pltpu.sync_copy(zer_hbm, a_sp.at[pl.ds(s * _ZR, _ZR), :])
    plsc.subcore_barrier()

    def load(j, b):
        return pltpu.async_copy(
            msg_hbm.at[pl.ds(row0 + j * _CP, _CP), :], bufs.at[b], ls[b])

    def drain_scat(b):
        pltpu.make_async_copy(bufs.at[b], a_sp.at[idx_v.at[0]], ss[b]).wait()

    def sgroup(g, carry):
        base = g * _NB
        lds = []
        for b in range(_NB):
            @pl.when(g > 0)
            def _(b=b):
                drain_scat(b)
            lds.append(load(base + b, b))
        for b in range(_NB):
            lds[b].wait()
            pltpu.async_copy(bufs.at[b], a_sp.at[idx_v.at[base + b]], ss[b],
                             add=True)
        return carry

    ngrp = _NF // _NB
    rem = _NF - ngrp * _NB
    lax.fori_loop(0, ngrp, sgroup, 0)
    for b in range(_NB):
        drain_scat(b)
    lds = [load(ngrp * _NB + b, b) for b in range(rem)]
    tl = pltpu.async_copy(msg_hbm.at[pl.ds(row0 + _NF * _CP, _TAIL), :],
                          bufs.at[rem, pl.ds(0, _TAIL), :], ls[rem])
    scs = []
    for b in range(rem):
        lds[b].wait()
        scs.append(pltpu.async_copy(bufs.at[b],
                                    a_sp.at[idx_v.at[ngrp * _NB + b]],
                                    ss[b], add=True))
    tl.wait()
    scs.append(pltpu.async_copy(bufs.at[rem], a_sp.at[idx_v.at[_NF]],
                                ss[rem], add=True))
    for d in scs:
        d.wait()
    plsc.subcore_barrier()

    for h in range(2):
        row0h = (2 * s + h) * _ET

        def gat(j, b, h=h):
            return pltpu.async_copy(a_sp.at[gidx_v.at[h, j]], bufs.at[b],
                                    ls[b])

        def stor(j, b, h=h):
            return pltpu.async_copy(
                bufs.at[b], g_hbm.at[c, pl.ds(row0h + j * _CP, _CP), :],
                ss[b])

        def drain_stor(b, h=h):
            pltpu.make_async_copy(
                bufs.at[b], g_hbm.at[c, pl.ds(row0h, _CP), :], ss[b]).wait()

        def ggroup(g, carry, h=h):
            base = g * _NB
            gts = []
            for b in range(_NB):
                @pl.when(g > 0)
                def _(b=b):
                    drain_stor(b)
                gts.append(gat(base + b, b))
            for b in range(_NB):
                gts[b].wait()
                stor(base + b, b)
            return carry

        lax.fori_loop(0, ngrp, ggroup, 0)
        for b in range(_NB):
            drain_stor(b)
        gts = [gat(ngrp * _NB + b, b) for b in range(rem)]
        gt = gat(_NF, rem)
        sts = []
        for b in range(rem):
            gts[b].wait()
            sts.append(stor(ngrp * _NB + b, b))
        gt.wait()
        sts.append(pltpu.async_copy(
            bufs.at[rem, pl.ds(0, _TAIL), :],
            g_hbm.at[c, pl.ds(row0h + _NF * _CP, _TAIL), :], ss[rem]))
        for d in sts:
            d.wait()


_sg_fused = pl.kernel(
    _sgf_body,
    out_type=jax.ShapeDtypeStruct((2, E, H), jnp.float32),
    mesh=_MESH,
    scratch_types=[
        pltpu.VMEM_SHARED((_NP, H), jnp.float32),
        pltpu.VMEM((_NCH, _CP), jnp.int32),
        pltpu.VMEM((2, _NCH, _CP), jnp.int32),
        pltpu.VMEM((_NB, _CP, H), jnp.float32),
    ] + [pltpu.SemaphoreType.DMA] * (2 * _NB),
)


def _gather_body(tbl_hbm, idx_hbm, g_hbm, idx_v, bufs, *sems):
    c = lax.axis_index("c")
    s = lax.axis_index("s")
    tid = c * _NS + s
    row0 = tid * _ET
    ls, ss = sems[:_NBG], sems[_NBG:]
    pltpu.sync_copy(idx_hbm.at[tid], idx_v)

    def gat(j, b):
        return pltpu.async_copy(tbl_hbm.at[idx_v.at[j]], bufs.at[b], ls[b])

    def stor(j, b):
        return pltpu.async_copy(
            bufs.at[b], g_hbm.at[pl.ds(row0 + j * _CP, _CP), :], ss[b])

    def drain_stor(b):
        pltpu.make_async_copy(
            bufs.at[b], g_hbm.at[pl.ds(row0, _CP), :], ss[b]).wait()

    def group(g, carry):
        base = g * _NBG
        gts = []
        for b in range(_NBG):
            @pl.when(g > 0)
            def _(b=b):
                drain_stor(b)
            gts.append(gat(base + b, b))
        for b in range(_NBG):
            gts[b].wait()
            stor(base + b, b)
        return carry

    ngrp = _NF // _NBG
    rem = _NF - ngrp * _NBG
    lax.fori_loop(0, ngrp, group, 0)
    for b in range(_NBG):
        drain_stor(b)
    gts = [gat(ngrp * _NBG + b, b) for b in range(rem)]
    gt = gat(_NF, rem)
    sts = []
    for b in range(rem):
        gts[b].wait()
        sts.append(stor(ngrp * _NBG + b, b))
    gt.wait()
    sts.append(pltpu.async_copy(
        bufs.at[rem, pl.ds(0, _TAIL), :],
        g_hbm.at[pl.ds(row0 + _NF * _CP, _TAIL), :], ss[rem]))
    for d in sts:
        d.wait()


_seg_gather = pl.kernel(
    _gather_body,
    out_type=jax.ShapeDtypeStruct((E, H), jnp.float32),
    mesh=_MESH,
    scratch_types=[
        pltpu.VMEM((_NCH, _CP), jnp.int32),
        pltpu.VMEM((_NBG, _CP, H), jnp.float32),
    ] + [pltpu.SemaphoreType.DMA] * (2 * _NBG),
)


def _gs_body(tbl_hbm, si_hbm, di_hbm, zer_hbm, p_hbm, a_sp, si_v, di_v, bufs,
             *sems):
    c = lax.axis_index("c")
    s = lax.axis_index("s")
    tid = c * _NS + s
    ls, ss = sems[:_NB], sems[_NB:]
    pltpu.sync_copy(si_hbm.at[tid], si_v)
    pltpu.sync_copy(di_hbm.at[tid], di_v)
    pltpu.sync_copy(zer_hbm, a_sp.at[pl.ds(s * _ZR, _ZR), :])
    plsc.subcore_barrier()

    def drain_scat(b):
        pltpu.make_async_copy(bufs.at[b], a_sp.at[di_v.at[0]], ss[b]).wait()

    def group(g, carry):
        base = g * _NB
        gts = []
        for b in range(_NB):
            @pl.when(g > 0)
            def _(b=b):
                drain_scat(b)
            gts.append(pltpu.async_copy(tbl_hbm.at[si_v.at[base + b]],
                                        bufs.at[b], ls[b]))
        for b in range(_NB):
            gts[b].wait()
            pltpu.async_copy(bufs.at[b], a_sp.at[di_v.at[base + b]], ss[b],
                             add=True)
        return carry

    lax.fori_loop(0, _NCH // _NB, group, 0)
    for b in range(_NB):
        drain_scat(b)
    plsc.subcore_barrier()
    pltpu.sync_copy(a_sp.at[pl.ds(s * _ZR, _ZR), :],
                    p_hbm.at[c, pl.ds(s * _ZR, _ZR), :])


_seg_gather_scatter = pl.kernel(
    _gs_body,
    out_type=jax.ShapeDtypeStruct((2, _NP, H), jnp.float32),
    mesh=_MESH,
    scratch_types=[
        pltpu.VMEM_SHARED((_NP, H), jnp.float32),
        pltpu.VMEM((_NCH, _CP), jnp.int32),
        pltpu.VMEM((_NCH, _CP), jnp.int32),
        pltpu.VMEM((_NB, _CP, H), jnp.float32),
    ] + [pltpu.SemaphoreType.DMA] * (2 * _NB),
)


def _readout_body(hid_hbm, idx_hbm, zer_hbm, one_hbm, sum_hbm, cnt_hbm,
                  s_sp, c_sp, idx_v, buf, obuf):
    c = lax.axis_index("c")
    s = lax.axis_index("s")
    tid = c * _NS + s
    row0 = tid * _AT
    pltpu.sync_copy(idx_hbm.at[tid], idx_v)
    pltpu.sync_copy(one_hbm, obuf)
    pltpu.sync_copy(zer_hbm.at[pl.ds(0, _MT), :],
                    s_sp.at[pl.ds(s * _MT, _MT), :])
    pltpu.sync_copy(zer_hbm.at[pl.ds(0, _MT), :],
                    c_sp.at[pl.ds(s * _MT, _MT), :])
    plsc.subcore_barrier()

    def sbody(j, carry):
        pltpu.sync_copy(hid_hbm.at[pl.ds(row0 + j * _CP, _CP), :], buf)
        pltpu.sync_copy(buf, s_sp.at[idx_v.at[j]], add=True)
        pltpu.sync_copy(obuf, c_sp.at[idx_v.at[j]], add=True)
        return carry

    lax.fori_loop(0, _AT // _CP, sbody, 0)
    plsc.subcore_barrier()
    pltpu.sync_copy(s_sp.at[pl.ds(s * _MT, _MT), :],
                    sum_hbm.at[c, pl.ds(s * _MT, _MT), :])
    pltpu.sync_copy(c_sp.at[pl.ds(s * _MT, _MT), :],
                    cnt_hbm.at[c, pl.ds(s * _MT, _MT), :])


_readout = pl.kernel(
    _readout_body,
    out_type=[jax.ShapeDtypeStruct((2, M, H), jnp.float32),
              jax.ShapeDtypeStruct((2, M, H), jnp.float32)],
    mesh=_MESH,
    scratch_types=[
        pltpu.VMEM_SHARED((M + 8, H), jnp.float32),
        pltpu.VMEM_SHARED((M + 8, H), jnp.float32),
        pltpu.VMEM((_AT // _CP, _CP), jnp.int32),
        pltpu.VMEM((_CP, H), jnp.float32),
        pltpu.VMEM((_CP, H), jnp.float32),
    ],
)


def kernel(f_atoms_r, f_bonds_r, f_atoms_p1, f_bonds_p1, f_atoms_p2,
           f_bonds_p2, W_i, b_i, W_h, b_h, W_o, b_o, Wd_i, bd_i, Wd_h, bd_h,
           Wd_o, bd_o, F1_W, F1_b, F2_W, F2_b, F3_W, F3_b, edge_index_r,
           b2revb_r, edge_index_p1, b2revb_p1, edge_index_p2, b2revb_p2,
           atom2mol, gpu):
    b_i2 = b_i.reshape(1, H)
    b_h2 = b_h.reshape(1, H)
    b_o2 = b_o.reshape(1, H)
    bd_i2 = bd_i.reshape(1, H)
    bd_h2 = bd_h.reshape(1, H)
    bd_o2 = bd_o.reshape(1, H)
    f1b = F1_b.reshape(1, H)
    f2b = F2_b.reshape(1, H)
    f3w = jnp.zeros((H, H), jnp.float32).at[:, :T].set(F3_W)
    f3b = jnp.zeros((1, H), jnp.float32).at[0, :T].set(F3_b)
    wo1, wo2 = W_o[:AF], W_o[AF:]
    wdo1, wdo2 = Wd_o[:H], Wd_o[H:]

    zer = jnp.zeros((_ZR, H), jnp.float32)
    one = jnp.ones((_CP, H), jnp.float32)

    def mpn(f_atoms, f_bonds, ei):
        idx3 = _pad_edge_idx(ei[1], N)
        inp, msg = _bonds_mm(f_bonds, W_i, b_i2)
        for _ in range(DEPTH - 1):
            g = _sg_fused(msg, idx3, zer)
            msg = _iter_mm(g, msg, W_h, inp, b_h2)
        am = _seg_scatter(msg, idx3, zer)
        return _out_mm(f_atoms, am, wo1, wo2, b_o2)

    r_atoms = mpn(f_atoms_r, f_bonds_r, edge_index_r)
    p1_atoms = mpn(f_atoms_p1, f_bonds_p1, edge_index_p1)
    p2_atoms = mpn(f_atoms_p2, f_bonds_p2, edge_index_p2)

    a2mp = jnp.concatenate(
        [atom2mol.astype(jnp.int32),
         jnp.full((_NHP - N,), M, jnp.int32)]).reshape(_NC * _NS,
                                                       _AT // _CP, _CP)

    def diff_head(ap, ei):
        si = _pad_edge_idx(ei[0], 0)
        di = _pad_edge_idx(ei[1], N)
        d, inp, msg = _diff_mm(ap, r_atoms, Wd_i, bd_i2)
        for _ in range(DEPTH - 1):
            agg = _seg_gather_scatter(msg, si, di, zer)
            msg = _diter_mm(agg, Wd_h, inp, bd_h2)
        hid = _hid_mm(d, msg, wdo1, wdo2, bd_o2, _NHP)
        return _readout(hid, a2mp, zer, one)

    s1, c1 = diff_head(p1_atoms, edge_index_p1)
    s2, _ = diff_head(p2_atoms, edge_index_p2)
    out = _ffn(s1, s2, c1, F1_W, f1b, F2_W, f2b, f3w, f3b)
    return out[:, :T]
